# baseline (speedup 1.0000x reference)
import functools

import jax
import jax.numpy as jnp
from jax import lax
from jax.experimental import pallas as pl
from jax.experimental.pallas import tpu as pltpu
from jax.experimental.pallas import tpu_sc as plsc

NC = 2
NS = 16
NW = NC * NS
BATCH = 64
GROUP = 40
NBUF = 4


def _mesh():
    return plsc.VectorSubcoreMesh(core_axis_name="c", subcore_axis_name="s")


def _hist_kernel(n_pad, nb):
    rpt = n_pad // NS

    @functools.partial(
        pl.kernel,
        out_type=jax.ShapeDtypeStruct((NC * n_pad,), jnp.float32),
        mesh=_mesh(),
        scratch_types=[
            pltpu.VMEM((nb, BATCH), jnp.int32),
            pltpu.VMEM((BATCH,), jnp.float32),
            pltpu.VMEM_SHARED((n_pad,), jnp.float32),
        ],
    )
    def hist(dst_hbm, zeros_hbm, out_hbm, idx_v, ones_v, deg_sp):
        cid = lax.axis_index("c")
        sid = lax.axis_index("s")
        wid = cid * NS + sid
        r0 = sid * rpt
        pltpu.sync_copy(zeros_hbm.at[pl.ds(r0, rpt)], deg_sp.at[pl.ds(r0, rpt)])
        for j in range(BATCH // 16):
            ones_v[pl.ds(j * 16, 16)] = jnp.full((16,), 1.0, jnp.float32)
        pltpu.sync_copy(dst_hbm.at[pl.ds(wid * nb, nb)], idx_v)
        plsc.subcore_barrier()

        def body(b, carry):
            pltpu.sync_copy(ones_v, deg_sp.at[idx_v.at[b]], add=True)
            return carry

        lax.fori_loop(0, nb, body, 0)
        plsc.subcore_barrier()
        pltpu.sync_copy(deg_sp.at[pl.ds(r0, rpt)],
                        out_hbm.at[pl.ds(cid * n_pad + r0, rpt)])

    return hist


def _scatter_kernel(n_pad, nb):
    rpt = n_pad // NS

    @functools.partial(
        pl.kernel,
        out_type=jax.ShapeDtypeStruct((NC * n_pad, 128), jnp.float32),
        mesh=_mesh(),
        scratch_types=[
            pltpu.VMEM((GROUP, BATCH), jnp.int32),
            pltpu.VMEM((GROUP, BATCH), jnp.int32),
            pltpu.VMEM((NBUF, BATCH, 128), jnp.float32),
            pltpu.VMEM_SHARED((n_pad, 128), jnp.float32),
            [pltpu.SemaphoreType.DMA] * NBUF,
            [pltpu.SemaphoreType.DMA] * NBUF,
        ],
    )
    def scat(h_hbm, src_hbm, dst_hbm, zeros_hbm, out_hbm,
             sidx_v, didx_v, rows_v, acc_sp, gsems, ssems):
        cid = lax.axis_index("c")
        sid = lax.axis_index("s")
        wid = cid * NS + sid
        r0 = sid * rpt
        pltpu.sync_copy(zeros_hbm.at[pl.ds(r0, rpt)], acc_sp.at[pl.ds(r0, rpt)])
        plsc.subcore_barrier()

        def group_body(g, carry):
            base = wid * nb + g * GROUP
            pltpu.sync_copy(src_hbm.at[pl.ds(base, GROUP)], sidx_v)
            pltpu.sync_copy(dst_hbm.at[pl.ds(base, GROUP)], didx_v)
            for k in range(2):
                pltpu.async_copy(h_hbm.at[sidx_v.at[k]], rows_v.at[k], gsems[k])

            def quad(i, c):
                for k in range(NBUF):
                    j = i * NBUF + k
                    pltpu.make_async_copy(
                        h_hbm.at[sidx_v.at[j]], rows_v.at[k], gsems[k]).wait()
                    pltpu.async_copy(rows_v.at[k], acc_sp.at[didx_v.at[j]],
                                     ssems[k], add=True)
                    kn = (k + 2) % NBUF

                    @pl.when(j >= 2)
                    def _():
                        pltpu.make_async_copy(
                            h_hbm.at[pl.ds(0, BATCH)], rows_v.at[kn],
                            ssems[kn]).wait()

                    @pl.when(j + 2 < GROUP)
                    def _():
                        pltpu.async_copy(
                            h_hbm.at[sidx_v.at[j + 2]], rows_v.at[kn],
                            gsems[kn])
                return c

            lax.fori_loop(0, GROUP // NBUF, quad, 0)
            for j in range(GROUP - 2, GROUP):
                pltpu.make_async_copy(
                    h_hbm.at[pl.ds(0, BATCH)], rows_v.at[j % NBUF],
                    ssems[j % NBUF]).wait()
            return carry

        lax.fori_loop(0, nb // GROUP, group_body, 0)
        plsc.subcore_barrier()
        pltpu.sync_copy(acc_sp.at[pl.ds(r0, rpt)],
                        out_hbm.at[pl.ds(cid * n_pad + r0, rpt)])

    return scat


def _matmul_scale(x_pad, w, deg_col, n_pad):
    bm = 1024

    def body(x_ref, w_ref, deg_ref, o_ref):
        dinv = lax.rsqrt(deg_ref[...])
        h = jnp.dot(x_ref[...], w_ref[...], preferred_element_type=jnp.float32)
        o_ref[...] = h * dinv

    return pl.pallas_call(
        body,
        grid=(n_pad // bm,),
        in_specs=[
            pl.BlockSpec((bm, 128), lambda i: (i, 0)),
            pl.BlockSpec((128, 128), lambda i: (0, 0)),
            pl.BlockSpec((bm, 1), lambda i: (i, 0)),
        ],
        out_specs=pl.BlockSpec((bm, 128), lambda i: (i, 0)),
        out_shape=jax.ShapeDtypeStruct((n_pad, 128), jnp.float32),
    )(x_pad, w, deg_col)


def _finalize(acc0, acc1, hprime, deg_col, b_row, n_pad):
    bm = 1024

    def body(a0_ref, a1_ref, h_ref, deg_ref, b_ref, o_ref):
        dinv = lax.rsqrt(deg_ref[...])
        s = a0_ref[...] + a1_ref[...] + h_ref[...]
        o_ref[...] = jnp.maximum(s * dinv + b_ref[...], 0.0)

    return pl.pallas_call(
        body,
        grid=(n_pad // bm,),
        in_specs=[
            pl.BlockSpec((bm, 128), lambda i: (i, 0)),
            pl.BlockSpec((bm, 128), lambda i: (i, 0)),
            pl.BlockSpec((bm, 128), lambda i: (i, 0)),
            pl.BlockSpec((bm, 1), lambda i: (i, 0)),
            pl.BlockSpec((1, 128), lambda i: (0, 0)),
        ],
        out_specs=pl.BlockSpec((bm, 128), lambda i: (i, 0)),
        out_shape=jax.ShapeDtypeStruct((n_pad, 128), jnp.float32),
    )(acc0, acc1, hprime, deg_col, b_row)


def kernel(x, edge_index, W, b):
    n, hidden = x.shape
    e = edge_index.shape[1]
    n_pad = -(-n // (NS * 128)) * (NS * 128)
    nb = -(-e // (NW * BATCH))
    nb = -(-nb // GROUP) * GROUP
    e_pad = NW * nb * BATCH

    src = edge_index[0].astype(jnp.int32)
    dst = edge_index[1].astype(jnp.int32)
    pad_idx = jnp.full((e_pad - e,), n, jnp.int32)
    src2 = jnp.concatenate([src, pad_idx]).reshape(NW * nb, BATCH)
    dst2 = jnp.concatenate([dst, pad_idx]).reshape(NW * nb, BATCH)

    x_pad = jnp.pad(x, ((0, n_pad - n), (0, 0)))
    zeros1 = jnp.zeros((n_pad,), jnp.float32)
    zeros2 = jnp.zeros((n_pad, 128), jnp.float32)

    hist = _hist_kernel(n_pad, nb)(dst2, zeros1)
    deg_col = (1.0 + hist[:n_pad] + hist[n_pad:]).reshape(n_pad, 1)

    hprime = _matmul_scale(x_pad, W, deg_col, n_pad)
    acc = _scatter_kernel(n_pad, nb)(hprime, src2, dst2, zeros2)
    out = _finalize(acc[:n_pad], acc[n_pad:], hprime, deg_col,
                    b.reshape(1, 128), n_pad)
    return out[:n]


# speedup vs baseline: 2.1837x; 2.1837x over previous
import functools

import jax
import jax.numpy as jnp
from jax import lax
from jax.experimental import pallas as pl
from jax.experimental.pallas import tpu as pltpu
from jax.experimental.pallas import tpu_sc as plsc

NC = 2
NS = 16
HALF = 64
BATCH = 128
GROUP = 40
NBUF = 2


def _mesh():
    return plsc.VectorSubcoreMesh(core_axis_name="c", subcore_axis_name="s")


def _hist_kernel(n_pad, nbh):
    rpt = n_pad // NS

    @functools.partial(
        pl.kernel,
        out_type=jax.ShapeDtypeStruct((NC * n_pad,), jnp.float32),
        mesh=_mesh(),
        scratch_types=[
            pltpu.VMEM((nbh, BATCH), jnp.int32),
            pltpu.VMEM((BATCH,), jnp.float32),
            pltpu.VMEM_SHARED((n_pad,), jnp.float32),
        ],
    )
    def hist(dst_hbm, zeros_hbm, out_hbm, idx_v, ones_v, deg_sp):
        cid = lax.axis_index("c")
        sid = lax.axis_index("s")
        wid = cid * NS + sid
        r0 = sid * rpt
        pltpu.sync_copy(zeros_hbm.at[pl.ds(r0, rpt)], deg_sp.at[pl.ds(r0, rpt)])
        for j in range(BATCH // 16):
            ones_v[pl.ds(j * 16, 16)] = jnp.full((16,), 1.0, jnp.float32)
        pltpu.sync_copy(dst_hbm.at[pl.ds(wid * nbh, nbh)], idx_v)
        plsc.subcore_barrier()

        def body(b, carry):
            pltpu.sync_copy(ones_v, deg_sp.at[idx_v.at[b]], add=True)
            return carry

        lax.fori_loop(0, nbh, body, 0)
        plsc.subcore_barrier()
        pltpu.sync_copy(deg_sp.at[pl.ds(r0, rpt)],
                        out_hbm.at[pl.ds(cid * n_pad + r0, rpt)])

    return hist


def _scatter_kernel(n_pad, nb):
    rpt = n_pad // NS

    @functools.partial(
        pl.kernel,
        out_type=jax.ShapeDtypeStruct((NC * n_pad, HALF), jnp.float32),
        mesh=_mesh(),
        compiler_params=pltpu.CompilerParams(use_tc_tiling_on_sc=False),
        scratch_types=[
            pltpu.VMEM((GROUP, BATCH), jnp.int32),
            pltpu.VMEM((GROUP, BATCH), jnp.int32),
            pltpu.VMEM((NBUF, BATCH, HALF), jnp.float32),
            pltpu.VMEM_SHARED((n_pad, HALF), jnp.float32),
            pltpu.VMEM_SHARED((n_pad, HALF), jnp.float32),
            [pltpu.SemaphoreType.DMA] * NBUF,
        ],
    )
    def scat(h0_hbm, h1_hbm, src_hbm, dst_hbm, zeros_hbm, out_hbm,
             sidx_v, didx_v, rows_v, h_sp, acc_sp, gsems):
        cid = lax.axis_index("c")
        sid = lax.axis_index("s")
        r0 = sid * rpt
        @pl.when(cid == 0)
        def _():
            pltpu.sync_copy(h0_hbm.at[pl.ds(r0, rpt)], h_sp.at[pl.ds(r0, rpt)])

        @pl.when(cid == 1)
        def _():
            pltpu.sync_copy(h1_hbm.at[pl.ds(r0, rpt)], h_sp.at[pl.ds(r0, rpt)])

        pltpu.sync_copy(zeros_hbm.at[pl.ds(r0, rpt)], acc_sp.at[pl.ds(r0, rpt)])
        plsc.subcore_barrier()

        def group_body(g, carry):
            base = sid * nb + g * GROUP
            pltpu.sync_copy(src_hbm.at[pl.ds(base, GROUP)], sidx_v)
            pltpu.sync_copy(dst_hbm.at[pl.ds(base, GROUP)], didx_v)
            for k in range(NBUF):
                pltpu.async_copy(h_sp.at[sidx_v.at[k]], rows_v.at[k], gsems[k])

            def pair(i, c):
                for k in range(NBUF):
                    j = i * NBUF + k
                    pltpu.make_async_copy(
                        h_sp.at[sidx_v.at[j]], rows_v.at[k], gsems[k]).wait()
                    pltpu.sync_copy(rows_v.at[k], acc_sp.at[didx_v.at[j]],
                                    add=True)

                    @pl.when(j + NBUF < GROUP)
                    def _():
                        pltpu.async_copy(
                            h_sp.at[sidx_v.at[j + NBUF]], rows_v.at[k],
                            gsems[k])
                return c

            lax.fori_loop(0, GROUP // NBUF, pair, 0)
            return carry

        lax.fori_loop(0, nb // GROUP, group_body, 0)
        plsc.subcore_barrier()
        pltpu.sync_copy(acc_sp.at[pl.ds(r0, rpt)],
                        out_hbm.at[pl.ds(cid * n_pad + r0, rpt)])

    return scat


def _matmul_scale(x_pad, w, deg_col, n_pad):
    bm = 1024

    def body(x_ref, w_ref, deg_ref, o_ref):
        dinv = lax.rsqrt(deg_ref[...])
        h = jnp.dot(x_ref[...], w_ref[...], preferred_element_type=jnp.float32)
        o_ref[...] = h * dinv

    return pl.pallas_call(
        body,
        grid=(n_pad // bm,),
        in_specs=[
            pl.BlockSpec((bm, 128), lambda i: (i, 0)),
            pl.BlockSpec((128, 128), lambda i: (0, 0)),
            pl.BlockSpec((bm, 1), lambda i: (i, 0)),
        ],
        out_specs=pl.BlockSpec((bm, 128), lambda i: (i, 0)),
        out_shape=jax.ShapeDtypeStruct((n_pad, 128), jnp.float32),
    )(x_pad, w, deg_col)


def _finalize(acc0, acc1, h0, h1, deg_col, b0, b1, n_pad):
    bm = 1024

    def body(a0_ref, a1_ref, h0_ref, h1_ref, deg_ref, b0_ref, b1_ref,
             o0_ref, o1_ref):
        dinv = lax.rsqrt(deg_ref[...])
        s0 = (a0_ref[...] + h0_ref[...]) * dinv + b0_ref[...]
        s1 = (a1_ref[...] + h1_ref[...]) * dinv + b1_ref[...]
        o0_ref[...] = jnp.maximum(s0, 0.0)
        o1_ref[...] = jnp.maximum(s1, 0.0)

    half_spec = pl.BlockSpec((bm, HALF), lambda i: (i, 0))
    return pl.pallas_call(
        body,
        grid=(n_pad // bm,),
        in_specs=[
            half_spec,
            half_spec,
            half_spec,
            half_spec,
            pl.BlockSpec((bm, 1), lambda i: (i, 0)),
            pl.BlockSpec((1, HALF), lambda i: (0, 0)),
            pl.BlockSpec((1, HALF), lambda i: (0, 0)),
        ],
        out_specs=[half_spec, half_spec],
        out_shape=[
            jax.ShapeDtypeStruct((n_pad, HALF), jnp.float32),
            jax.ShapeDtypeStruct((n_pad, HALF), jnp.float32),
        ],
    )(acc0, acc1, h0, h1, deg_col, b0, b1)


def kernel(x, edge_index, W, b):
    n, hidden = x.shape
    e = edge_index.shape[1]
    n_pad = -(-n // (NS * 128)) * (NS * 128)

    src = edge_index[0].astype(jnp.int32)
    dst = edge_index[1].astype(jnp.int32)

    nbh = -(-e // (NC * NS * BATCH))
    nbh = -(-nbh // 8) * 8
    e_pad_h = NC * NS * nbh * BATCH
    pad_h = jnp.full((e_pad_h - e,), n, jnp.int32)
    dst2h = jnp.concatenate([dst, pad_h]).reshape(NC * NS * nbh, BATCH)

    nb = -(-e // (NS * BATCH))
    nb = -(-nb // GROUP) * GROUP
    e_pad = NS * nb * BATCH
    pad_idx = jnp.full((e_pad - e,), n, jnp.int32)
    src2 = jnp.concatenate([src, pad_idx]).reshape(NS * nb, BATCH)
    dst2 = jnp.concatenate([dst, pad_idx]).reshape(NS * nb, BATCH)

    x_pad = jnp.pad(x, ((0, n_pad - n), (0, 0)))
    zeros1 = jnp.zeros((n_pad,), jnp.float32)
    zeros2 = jnp.zeros((n_pad, HALF), jnp.float32)

    hist = _hist_kernel(n_pad, nbh)(dst2h, zeros1)
    deg_col = (1.0 + hist[:n_pad] + hist[n_pad:]).reshape(n_pad, 1)

    hp = _matmul_scale(x_pad, W, deg_col, n_pad)
    h0, h1 = hp[:, :HALF], hp[:, HALF:]
    acc = _scatter_kernel(n_pad, nb)(h0, h1, src2, dst2, zeros2)
    o0, o1 = _finalize(acc[:n_pad], acc[n_pad:], h0, h1, deg_col,
                       b[:HALF].reshape(1, HALF), b[HALF:].reshape(1, HALF),
                       n_pad)
    return jnp.concatenate([o0, o1], axis=1)[:n]


# speedup vs baseline: 2.5009x; 1.1452x over previous
import functools

import jax
import jax.numpy as jnp
from jax import lax
from jax.experimental import pallas as pl
from jax.experimental.pallas import tpu as pltpu
from jax.experimental.pallas import tpu_sc as plsc

NC = 2
NS = 16
HALF = 64
BATCH = 128
GROUP = 40
NBUF = 2


def _mesh():
    return plsc.VectorSubcoreMesh(core_axis_name="c", subcore_axis_name="s")


def _hist_kernel(n_pad, nbh):
    rpt = n_pad // NS

    @functools.partial(
        pl.kernel,
        out_type=jax.ShapeDtypeStruct((NC * n_pad,), jnp.float32),
        mesh=_mesh(),
        scratch_types=[
            pltpu.VMEM((nbh, BATCH), jnp.int32),
            pltpu.VMEM((BATCH,), jnp.float32),
            pltpu.VMEM_SHARED((n_pad,), jnp.float32),
        ],
    )
    def hist(dst_hbm, zeros_hbm, out_hbm, idx_v, ones_v, deg_sp):
        cid = lax.axis_index("c")
        sid = lax.axis_index("s")
        wid = sid * NC + cid
        r0 = sid * rpt
        pltpu.sync_copy(zeros_hbm.at[pl.ds(r0, rpt)], deg_sp.at[pl.ds(r0, rpt)])
        for j in range(BATCH // 16):
            ones_v[pl.ds(j * 16, 16)] = jnp.full((16,), 1.0, jnp.float32)
        pltpu.sync_copy(dst_hbm.at[pl.ds(wid * nbh, nbh)], idx_v)
        plsc.subcore_barrier()

        def body(b, carry):
            pltpu.sync_copy(ones_v, deg_sp.at[idx_v.at[b]], add=True)
            return carry

        lax.fori_loop(0, nbh, body, 0)
        plsc.subcore_barrier()
        pltpu.sync_copy(deg_sp.at[pl.ds(r0, rpt)],
                        out_hbm.at[pl.ds(cid * n_pad + r0, rpt)])

    return hist


def _scatter_kernel(n_pad, nb):
    rpt = n_pad // NS

    @functools.partial(
        pl.kernel,
        out_type=jax.ShapeDtypeStruct((n_pad, 128), jnp.float32),
        mesh=_mesh(),
        compiler_params=pltpu.CompilerParams(use_tc_tiling_on_sc=False),
        scratch_types=[
            pltpu.VMEM((GROUP, BATCH), jnp.int32),
            pltpu.VMEM((GROUP, BATCH), jnp.int32),
            pltpu.VMEM((NBUF, BATCH, HALF), jnp.float32),
            pltpu.VMEM_SHARED((n_pad, HALF), jnp.float32),
            pltpu.VMEM_SHARED((n_pad, HALF), jnp.float32),
            [pltpu.SemaphoreType.DMA] * NBUF,
        ],
    )
    def scat(hp_hbm, src_hbm, dst_hbm, zeros_hbm, out_hbm,
             sidx_v, didx_v, rows_v, h_sp, acc_sp, gsems):
        cid = lax.axis_index("c")
        sid = lax.axis_index("s")
        r0 = sid * rpt
        pltpu.sync_copy(hp_hbm.at[pl.ds(r0, rpt), pl.ds(cid * HALF, HALF)],
                        h_sp.at[pl.ds(r0, rpt)])
        pltpu.sync_copy(zeros_hbm.at[pl.ds(r0, rpt)], acc_sp.at[pl.ds(r0, rpt)])
        plsc.subcore_barrier()

        def group_body(g, carry):
            base = sid * nb + g * GROUP
            pltpu.sync_copy(src_hbm.at[pl.ds(base, GROUP)], sidx_v)
            pltpu.sync_copy(dst_hbm.at[pl.ds(base, GROUP)], didx_v)
            for k in range(NBUF):
                pltpu.async_copy(h_sp.at[sidx_v.at[k]], rows_v.at[k], gsems[k])

            def pair(i, c):
                for k in range(NBUF):
                    j = i * NBUF + k
                    pltpu.make_async_copy(
                        h_sp.at[sidx_v.at[j]], rows_v.at[k], gsems[k]).wait()
                    pltpu.sync_copy(rows_v.at[k], acc_sp.at[didx_v.at[j]],
                                    add=True)

                    @pl.when(j + NBUF < GROUP)
                    def _():
                        pltpu.async_copy(
                            h_sp.at[sidx_v.at[j + NBUF]], rows_v.at[k],
                            gsems[k])
                return c

            lax.fori_loop(0, GROUP // NBUF, pair, 0)
            return carry

        lax.fori_loop(0, nb // GROUP, group_body, 0)
        plsc.subcore_barrier()
        pltpu.sync_copy(acc_sp.at[pl.ds(r0, rpt)],
                        out_hbm.at[pl.ds(r0, rpt), pl.ds(cid * HALF, HALF)])

    return scat


def _matmul_scale(x_pad, w, deg_col, n_pad):
    bm = 1024

    def body(x_ref, w_ref, deg_ref, o_ref):
        dinv = lax.rsqrt(deg_ref[...])
        h = jnp.dot(x_ref[...], w_ref[...], preferred_element_type=jnp.float32)
        o_ref[...] = h * dinv

    return pl.pallas_call(
        body,
        grid=(n_pad // bm,),
        in_specs=[
            pl.BlockSpec((bm, 128), lambda i: (i, 0)),
            pl.BlockSpec((128, 128), lambda i: (0, 0)),
            pl.BlockSpec((bm, 1), lambda i: (i, 0)),
        ],
        out_specs=pl.BlockSpec((bm, 128), lambda i: (i, 0)),
        out_shape=jax.ShapeDtypeStruct((n_pad, 128), jnp.float32),
    )(x_pad, w, deg_col)


def _finalize(acc, hp, deg_col, b_row, n_pad):
    bm = 1024

    def body(a_ref, h_ref, deg_ref, b_ref, o_ref):
        dinv = lax.rsqrt(deg_ref[...])
        s = (a_ref[...] + h_ref[...]) * dinv + b_ref[...]
        o_ref[...] = jnp.maximum(s, 0.0)

    full_spec = pl.BlockSpec((bm, 128), lambda i: (i, 0))
    return pl.pallas_call(
        body,
        grid=(n_pad // bm,),
        in_specs=[
            full_spec,
            full_spec,
            pl.BlockSpec((bm, 1), lambda i: (i, 0)),
            pl.BlockSpec((1, 128), lambda i: (0, 0)),
        ],
        out_specs=full_spec,
        out_shape=jax.ShapeDtypeStruct((n_pad, 128), jnp.float32),
    )(acc, hp, deg_col, b_row)


def kernel(x, edge_index, W, b):
    n, hidden = x.shape
    e = edge_index.shape[1]
    n_pad = -(-n // (NS * 128)) * (NS * 128)

    src = edge_index[0].astype(jnp.int32)
    dst = edge_index[1].astype(jnp.int32)

    nb = -(-e // (NS * BATCH))
    nb = -(-nb // GROUP) * GROUP
    e_pad = NS * nb * BATCH
    pad_idx = jnp.full((e_pad - e,), n, jnp.int32)
    src2 = jnp.concatenate([src, pad_idx]).reshape(NS * nb, BATCH)
    dst2 = jnp.concatenate([dst, pad_idx]).reshape(NS * nb, BATCH)

    x_pad = jnp.pad(x, ((0, n_pad - n), (0, 0)))
    zeros1 = jnp.zeros((n_pad,), jnp.float32)
    zeros2 = jnp.zeros((n_pad, HALF), jnp.float32)

    hist = _hist_kernel(n_pad, nb // 2)(dst2, zeros1)
    deg_col = (1.0 + hist[:n_pad] + hist[n_pad:]).reshape(n_pad, 1)

    hp = _matmul_scale(x_pad, W, deg_col, n_pad)
    acc = _scatter_kernel(n_pad, nb)(hp, src2, dst2, zeros2)
    out = _finalize(acc, hp, deg_col, b.reshape(1, 128), n_pad)
    return out[:n]


# speedup vs baseline: 2.8817x; 1.1523x over previous
import functools

import jax
import jax.numpy as jnp
from jax import lax
from jax.experimental import pallas as pl
from jax.experimental.pallas import tpu as pltpu
from jax.experimental.pallas import tpu_sc as plsc

NC = 2
NS = 16
HALF = 64
BATCH = 128
GROUP = 40
NBUF = 4


def _mesh():
    return plsc.VectorSubcoreMesh(core_axis_name="c", subcore_axis_name="s")


def _hist_kernel(n_pad, nbh):
    rpt = n_pad // NS

    @functools.partial(
        pl.kernel,
        out_type=jax.ShapeDtypeStruct((NC * n_pad,), jnp.float32),
        mesh=_mesh(),
        scratch_types=[
            pltpu.VMEM((nbh, BATCH), jnp.int32),
            pltpu.VMEM((BATCH,), jnp.float32),
            pltpu.VMEM_SHARED((n_pad,), jnp.float32),
        ],
    )
    def hist(dst_hbm, zeros_hbm, out_hbm, idx_v, ones_v, deg_sp):
        cid = lax.axis_index("c")
        sid = lax.axis_index("s")
        wid = sid * NC + cid
        r0 = sid * rpt
        pltpu.sync_copy(zeros_hbm.at[pl.ds(r0, rpt)], deg_sp.at[pl.ds(r0, rpt)])
        for j in range(BATCH // 16):
            ones_v[pl.ds(j * 16, 16)] = jnp.full((16,), 1.0, jnp.float32)
        pltpu.sync_copy(dst_hbm.at[pl.ds(wid * nbh, nbh)], idx_v)
        plsc.subcore_barrier()

        def body(b, carry):
            pltpu.sync_copy(ones_v, deg_sp.at[idx_v.at[b]], add=True)
            return carry

        lax.fori_loop(0, nbh, body, 0)
        plsc.subcore_barrier()
        pltpu.sync_copy(deg_sp.at[pl.ds(r0, rpt)],
                        out_hbm.at[pl.ds(cid * n_pad + r0, rpt)])

    return hist


def _scatter_kernel(n_pad, nb):
    rpt = n_pad // NS

    @functools.partial(
        pl.kernel,
        out_type=jax.ShapeDtypeStruct((n_pad, 128), jnp.float32),
        mesh=_mesh(),
        compiler_params=pltpu.CompilerParams(use_tc_tiling_on_sc=False),
        scratch_types=[
            pltpu.VMEM((GROUP, BATCH), jnp.int32),
            pltpu.VMEM((GROUP, BATCH), jnp.int32),
            pltpu.VMEM((NBUF, BATCH, HALF), jnp.float32),
            pltpu.VMEM_SHARED((n_pad, HALF), jnp.float32),
            pltpu.VMEM_SHARED((n_pad, HALF), jnp.float32),
            [pltpu.SemaphoreType.DMA] * NBUF,
            [pltpu.SemaphoreType.DMA] * NBUF,
        ],
    )
    def scat(hp_hbm, src_hbm, dst_hbm, zeros_hbm, out_hbm,
             sidx_v, didx_v, rows_v, h_sp, acc_sp, gsems, ssems):
        cid = lax.axis_index("c")
        sid = lax.axis_index("s")
        r0 = sid * rpt
        pltpu.sync_copy(hp_hbm.at[pl.ds(r0, rpt), pl.ds(cid * HALF, HALF)],
                        h_sp.at[pl.ds(r0, rpt)])
        pltpu.sync_copy(zeros_hbm.at[pl.ds(r0, rpt)], acc_sp.at[pl.ds(r0, rpt)])
        plsc.subcore_barrier()

        def group_body(g, carry):
            base = sid * nb + g * GROUP
            pltpu.sync_copy(src_hbm.at[pl.ds(base, GROUP)], sidx_v)
            pltpu.sync_copy(dst_hbm.at[pl.ds(base, GROUP)], didx_v)
            for k in range(2):
                pltpu.async_copy(h_sp.at[sidx_v.at[k]], rows_v.at[k], gsems[k])

            def quad(i, c):
                for k in range(NBUF):
                    j = i * NBUF + k
                    pltpu.make_async_copy(
                        h_sp.at[sidx_v.at[j]], rows_v.at[k], gsems[k]).wait()
                    pltpu.async_copy(rows_v.at[k], acc_sp.at[didx_v.at[j]],
                                     ssems[k], add=True)
                    kn = (k + 2) % NBUF

                    @pl.when(j >= 2)
                    def _():
                        pltpu.make_async_copy(
                            zeros_hbm.at[pl.ds(0, BATCH)], rows_v.at[kn],
                            ssems[kn]).wait()

                    @pl.when(j + 2 < GROUP)
                    def _():
                        pltpu.async_copy(
                            h_sp.at[sidx_v.at[j + 2]], rows_v.at[kn],
                            gsems[kn])
                return c

            lax.fori_loop(0, GROUP // NBUF, quad, 0)
            for j in range(GROUP - 2, GROUP):
                pltpu.make_async_copy(
                    zeros_hbm.at[pl.ds(0, BATCH)], rows_v.at[j % NBUF],
                    ssems[j % NBUF]).wait()
            return carry

        lax.fori_loop(0, nb // GROUP, group_body, 0)
        plsc.subcore_barrier()
        pltpu.sync_copy(acc_sp.at[pl.ds(r0, rpt)],
                        out_hbm.at[pl.ds(r0, rpt), pl.ds(cid * HALF, HALF)])

    return scat


def _matmul_scale(x_pad, w, deg_col, n_pad):
    bm = 1024

    def body(x_ref, w_ref, deg_ref, o_ref):
        dinv = lax.rsqrt(deg_ref[...])
        h = jnp.dot(x_ref[...], w_ref[...], preferred_element_type=jnp.float32)
        o_ref[...] = h * dinv

    return pl.pallas_call(
        body,
        grid=(n_pad // bm,),
        in_specs=[
            pl.BlockSpec((bm, 128), lambda i: (i, 0)),
            pl.BlockSpec((128, 128), lambda i: (0, 0)),
            pl.BlockSpec((bm, 1), lambda i: (i, 0)),
        ],
        out_specs=pl.BlockSpec((bm, 128), lambda i: (i, 0)),
        out_shape=jax.ShapeDtypeStruct((n_pad, 128), jnp.float32),
    )(x_pad, w, deg_col)


def _finalize(acc, hp, deg_col, b_row, n_pad):
    bm = 1024

    def body(a_ref, h_ref, deg_ref, b_ref, o_ref):
        dinv = lax.rsqrt(deg_ref[...])
        s = (a_ref[...] + h_ref[...]) * dinv + b_ref[...]
        o_ref[...] = jnp.maximum(s, 0.0)

    full_spec = pl.BlockSpec((bm, 128), lambda i: (i, 0))
    return pl.pallas_call(
        body,
        grid=(n_pad // bm,),
        in_specs=[
            full_spec,
            full_spec,
            pl.BlockSpec((bm, 1), lambda i: (i, 0)),
            pl.BlockSpec((1, 128), lambda i: (0, 0)),
        ],
        out_specs=full_spec,
        out_shape=jax.ShapeDtypeStruct((n_pad, 128), jnp.float32),
    )(acc, hp, deg_col, b_row)


def kernel(x, edge_index, W, b):
    n, hidden = x.shape
    e = edge_index.shape[1]
    n_pad = -(-n // (NS * 128)) * (NS * 128)

    src = edge_index[0].astype(jnp.int32)
    dst = edge_index[1].astype(jnp.int32)

    nb = -(-e // (NS * BATCH))
    nb = -(-nb // GROUP) * GROUP
    e_pad = NS * nb * BATCH
    pad_idx = jnp.full((e_pad - e,), n, jnp.int32)
    src2 = jnp.concatenate([src, pad_idx]).reshape(NS * nb, BATCH)
    dst2 = jnp.concatenate([dst, pad_idx]).reshape(NS * nb, BATCH)

    x_pad = jnp.pad(x, ((0, n_pad - n), (0, 0)))
    zeros1 = jnp.zeros((n_pad,), jnp.float32)
    zeros2 = jnp.zeros((n_pad, HALF), jnp.float32)

    hist = _hist_kernel(n_pad, nb // 2)(dst2, zeros1)
    deg_col = (1.0 + hist[:n_pad] + hist[n_pad:]).reshape(n_pad, 1)

    hp = _matmul_scale(x_pad, W, deg_col, n_pad)
    acc = _scatter_kernel(n_pad, nb)(hp, src2, dst2, zeros2)
    out = _finalize(acc, hp, deg_col, b.reshape(1, 128), n_pad)
    return out[:n]


# speedup vs baseline: 3.4096x; 1.1832x over previous
import functools

import jax
import jax.numpy as jnp
from jax import lax
from jax.experimental import pallas as pl
from jax.experimental.pallas import tpu as pltpu
from jax.experimental.pallas import tpu_sc as plsc

NC = 2
NS = 16
HALF = 64
BATCH = 128
GROUP = 40
NBUF = 4


def _mesh():
    return plsc.VectorSubcoreMesh(core_axis_name="c", subcore_axis_name="s")


def _hist_kernel(n_pad, nbh):
    rpt = n_pad // NS

    @functools.partial(
        pl.kernel,
        out_type=jax.ShapeDtypeStruct((NC * n_pad,), jnp.float32),
        mesh=_mesh(),
        scratch_types=[
            pltpu.VMEM((nbh, BATCH), jnp.int32),
            pltpu.VMEM((BATCH,), jnp.float32),
            pltpu.VMEM_SHARED((n_pad,), jnp.float32),
        ],
    )
    def hist(dst_hbm, zeros_hbm, out_hbm, idx_v, ones_v, deg_sp):
        cid = lax.axis_index("c")
        sid = lax.axis_index("s")
        wid = sid * NC + cid
        r0 = sid * rpt
        pltpu.sync_copy(zeros_hbm.at[pl.ds(r0, rpt)], deg_sp.at[pl.ds(r0, rpt)])
        for j in range(BATCH // 16):
            ones_v[pl.ds(j * 16, 16)] = jnp.full((16,), 1.0, jnp.float32)
        pltpu.sync_copy(dst_hbm.at[pl.ds(wid * nbh, nbh)], idx_v)
        plsc.subcore_barrier()

        def body(b, carry):
            pltpu.sync_copy(ones_v, deg_sp.at[idx_v.at[b]], add=True)
            return carry

        lax.fori_loop(0, nbh, body, 0)
        plsc.subcore_barrier()
        pltpu.sync_copy(deg_sp.at[pl.ds(r0, rpt)],
                        out_hbm.at[pl.ds(cid * n_pad + r0, rpt)])

    return hist


def _scatter_kernel(n_pad, nb):
    rpt = n_pad // NS

    @functools.partial(
        pl.kernel,
        out_type=jax.ShapeDtypeStruct((n_pad, 128), jnp.bfloat16),
        mesh=_mesh(),
        compiler_params=pltpu.CompilerParams(use_tc_tiling_on_sc=False),
        scratch_types=[
            pltpu.VMEM((GROUP, BATCH), jnp.int32),
            pltpu.VMEM((GROUP, BATCH), jnp.int32),
            pltpu.VMEM((NBUF, BATCH, HALF), jnp.bfloat16),
            pltpu.VMEM_SHARED((n_pad, HALF), jnp.bfloat16),
            pltpu.VMEM_SHARED((n_pad, HALF), jnp.bfloat16),
            [pltpu.SemaphoreType.DMA] * NBUF,
            [pltpu.SemaphoreType.DMA] * NBUF,
        ],
    )
    def scat(hp_hbm, src_hbm, dst_hbm, zeros_hbm, out_hbm,
             sidx_v, didx_v, rows_v, h_sp, acc_sp, gsems, ssems):
        cid = lax.axis_index("c")
        sid = lax.axis_index("s")
        r0 = sid * rpt
        pltpu.sync_copy(hp_hbm.at[pl.ds(r0, rpt), pl.ds(cid * HALF, HALF)],
                        h_sp.at[pl.ds(r0, rpt)])
        pltpu.sync_copy(zeros_hbm.at[pl.ds(r0, rpt)], acc_sp.at[pl.ds(r0, rpt)])
        plsc.subcore_barrier()

        def group_body(g, carry):
            base = sid * nb + g * GROUP
            pltpu.sync_copy(src_hbm.at[pl.ds(base, GROUP)], sidx_v)
            pltpu.sync_copy(dst_hbm.at[pl.ds(base, GROUP)], didx_v)
            for k in range(2):
                pltpu.async_copy(h_sp.at[sidx_v.at[k]], rows_v.at[k], gsems[k])

            def quad(i, c):
                for k in range(NBUF):
                    j = i * NBUF + k
                    pltpu.make_async_copy(
                        h_sp.at[sidx_v.at[j]], rows_v.at[k], gsems[k]).wait()
                    pltpu.async_copy(rows_v.at[k], acc_sp.at[didx_v.at[j]],
                                     ssems[k], add=True)
                    kn = (k + 2) % NBUF

                    @pl.when(j >= 2)
                    def _():
                        pltpu.make_async_copy(
                            zeros_hbm.at[pl.ds(0, BATCH)], rows_v.at[kn],
                            ssems[kn]).wait()

                    @pl.when(j + 2 < GROUP)
                    def _():
                        pltpu.async_copy(
                            h_sp.at[sidx_v.at[j + 2]], rows_v.at[kn],
                            gsems[kn])
                return c

            lax.fori_loop(0, GROUP // NBUF, quad, 0)
            for j in range(GROUP - 2, GROUP):
                pltpu.make_async_copy(
                    zeros_hbm.at[pl.ds(0, BATCH)], rows_v.at[j % NBUF],
                    ssems[j % NBUF]).wait()
            return carry

        lax.fori_loop(0, nb // GROUP, group_body, 0)
        plsc.subcore_barrier()
        pltpu.sync_copy(acc_sp.at[pl.ds(r0, rpt)],
                        out_hbm.at[pl.ds(r0, rpt), pl.ds(cid * HALF, HALF)])

    return scat


def _matmul_scale(x_pad, w, deg_col, n_pad):
    bm = 1024

    def body(x_ref, w_ref, deg_ref, o_ref, o16_ref):
        dinv = lax.rsqrt(deg_ref[...])
        h = jnp.dot(x_ref[...], w_ref[...], preferred_element_type=jnp.float32)
        hs = h * dinv
        o_ref[...] = hs
        o16_ref[...] = hs.astype(jnp.bfloat16)

    return pl.pallas_call(
        body,
        grid=(n_pad // bm,),
        in_specs=[
            pl.BlockSpec((bm, 128), lambda i: (i, 0)),
            pl.BlockSpec((128, 128), lambda i: (0, 0)),
            pl.BlockSpec((bm, 1), lambda i: (i, 0)),
        ],
        out_specs=[
            pl.BlockSpec((bm, 128), lambda i: (i, 0)),
            pl.BlockSpec((bm, 128), lambda i: (i, 0)),
        ],
        out_shape=[
            jax.ShapeDtypeStruct((n_pad, 128), jnp.float32),
            jax.ShapeDtypeStruct((n_pad, 128), jnp.bfloat16),
        ],
    )(x_pad, w, deg_col)


def _finalize(acc, hp, deg_col, b_row, n_pad):
    bm = 1024

    def body(a_ref, h_ref, deg_ref, b_ref, o_ref):
        dinv = lax.rsqrt(deg_ref[...])
        s = (a_ref[...].astype(jnp.float32) + h_ref[...]) * dinv + b_ref[...]
        o_ref[...] = jnp.maximum(s, 0.0)

    full_spec = pl.BlockSpec((bm, 128), lambda i: (i, 0))
    return pl.pallas_call(
        body,
        grid=(n_pad // bm,),
        in_specs=[
            full_spec,
            full_spec,
            pl.BlockSpec((bm, 1), lambda i: (i, 0)),
            pl.BlockSpec((1, 128), lambda i: (0, 0)),
        ],
        out_specs=full_spec,
        out_shape=jax.ShapeDtypeStruct((n_pad, 128), jnp.float32),
    )(acc, hp, deg_col, b_row)


def kernel(x, edge_index, W, b):
    n, hidden = x.shape
    e = edge_index.shape[1]
    n_pad = -(-n // (NS * 128)) * (NS * 128)

    src = edge_index[0].astype(jnp.int32)
    dst = edge_index[1].astype(jnp.int32)

    nb = -(-e // (NS * BATCH))
    nb = -(-nb // GROUP) * GROUP
    e_pad = NS * nb * BATCH
    pad_idx = jnp.full((e_pad - e,), n, jnp.int32)
    src2 = jnp.concatenate([src, pad_idx]).reshape(NS * nb, BATCH)
    dst2 = jnp.concatenate([dst, pad_idx]).reshape(NS * nb, BATCH)

    x_pad = jnp.pad(x, ((0, n_pad - n), (0, 0)))
    zeros1 = jnp.zeros((n_pad,), jnp.float32)
    zeros2 = jnp.zeros((n_pad, HALF), jnp.bfloat16)

    hist = _hist_kernel(n_pad, nb // 2)(dst2, zeros1)
    deg_col = (1.0 + hist[:n_pad] + hist[n_pad:]).reshape(n_pad, 1)

    hp, hp16 = _matmul_scale(x_pad, W, deg_col, n_pad)
    acc = _scatter_kernel(n_pad, nb)(hp16, src2, dst2, zeros2)
    out = _finalize(acc, hp, deg_col, b.reshape(1, 128), n_pad)
    return out[:n]


# speedup vs baseline: 3.4947x; 1.0249x over previous
import functools

import jax
import jax.numpy as jnp
from jax import lax
from jax.experimental import pallas as pl
from jax.experimental.pallas import tpu as pltpu
from jax.experimental.pallas import tpu_sc as plsc

NC = 2
NS = 16
HALF = 64
BATCH = 128
GROUP = 160
NBUF = 4
AHEAD = 2


def _mesh():
    return plsc.VectorSubcoreMesh(core_axis_name="c", subcore_axis_name="s")


def _hist_kernel(n_pad, nbh):
    rpt = n_pad // NS

    @functools.partial(
        pl.kernel,
        out_type=jax.ShapeDtypeStruct((NC * n_pad,), jnp.float32),
        mesh=_mesh(),
        scratch_types=[
            pltpu.VMEM((nbh, BATCH), jnp.int32),
            pltpu.VMEM((BATCH,), jnp.float32),
            pltpu.VMEM_SHARED((n_pad,), jnp.float32),
        ],
    )
    def hist(dst_hbm, zeros_hbm, out_hbm, idx_v, ones_v, deg_sp):
        cid = lax.axis_index("c")
        sid = lax.axis_index("s")
        wid = sid * NC + cid
        r0 = sid * rpt
        pltpu.sync_copy(zeros_hbm.at[pl.ds(r0, rpt)], deg_sp.at[pl.ds(r0, rpt)])
        for j in range(BATCH // 16):
            ones_v[pl.ds(j * 16, 16)] = jnp.full((16,), 1.0, jnp.float32)
        pltpu.sync_copy(dst_hbm.at[pl.ds(wid * nbh, nbh)], idx_v)
        plsc.subcore_barrier()

        def body(b, carry):
            pltpu.sync_copy(ones_v, deg_sp.at[idx_v.at[b]], add=True)
            return carry

        lax.fori_loop(0, nbh, body, 0)
        plsc.subcore_barrier()
        pltpu.sync_copy(deg_sp.at[pl.ds(r0, rpt)],
                        out_hbm.at[pl.ds(cid * n_pad + r0, rpt)])

    return hist


def _scatter_kernel(n_pad, nb):
    rpt = n_pad // NS

    @functools.partial(
        pl.kernel,
        out_type=jax.ShapeDtypeStruct((n_pad, 128), jnp.bfloat16),
        mesh=_mesh(),
        compiler_params=pltpu.CompilerParams(use_tc_tiling_on_sc=False),
        scratch_types=[
            pltpu.VMEM((GROUP, BATCH), jnp.int32),
            pltpu.VMEM((GROUP, BATCH), jnp.int32),
            pltpu.VMEM((NBUF, BATCH, HALF), jnp.bfloat16),
            pltpu.VMEM_SHARED((n_pad, HALF), jnp.bfloat16),
            pltpu.VMEM_SHARED((n_pad, HALF), jnp.bfloat16),
            [pltpu.SemaphoreType.DMA] * NBUF,
            [pltpu.SemaphoreType.DMA] * NBUF,
        ],
    )
    def scat(hp_hbm, src_hbm, dst_hbm, zeros_hbm, out_hbm,
             sidx_v, didx_v, rows_v, h_sp, acc_sp, gsems, ssems):
        cid = lax.axis_index("c")
        sid = lax.axis_index("s")
        r0 = sid * rpt
        pltpu.sync_copy(hp_hbm.at[pl.ds(r0, rpt), pl.ds(cid * HALF, HALF)],
                        h_sp.at[pl.ds(r0, rpt)])
        pltpu.sync_copy(zeros_hbm.at[pl.ds(r0, rpt)], acc_sp.at[pl.ds(r0, rpt)])
        plsc.subcore_barrier()

        def group_body(g, carry):
            base = sid * nb + g * GROUP
            pltpu.sync_copy(src_hbm.at[pl.ds(base, GROUP)], sidx_v)
            pltpu.sync_copy(dst_hbm.at[pl.ds(base, GROUP)], didx_v)
            for k in range(AHEAD):
                pltpu.async_copy(h_sp.at[sidx_v.at[k]], rows_v.at[k], gsems[k])

            def ring(i, c):
                for k in range(NBUF):
                    j = i * NBUF + k
                    pltpu.make_async_copy(
                        h_sp.at[sidx_v.at[j]], rows_v.at[k], gsems[k]).wait()
                    pltpu.async_copy(rows_v.at[k], acc_sp.at[didx_v.at[j]],
                                     ssems[k], add=True)
                    kn = (k + AHEAD) % NBUF

                    @pl.when(j + AHEAD >= NBUF)
                    def _():
                        pltpu.make_async_copy(
                            zeros_hbm.at[pl.ds(0, BATCH)], rows_v.at[kn],
                            ssems[kn]).wait()

                    @pl.when(j + AHEAD < GROUP)
                    def _():
                        pltpu.async_copy(
                            h_sp.at[sidx_v.at[j + AHEAD]], rows_v.at[kn],
                            gsems[kn])
                return c

            lax.fori_loop(0, GROUP // NBUF, ring, 0)
            for j in range(GROUP + AHEAD - NBUF, GROUP):
                pltpu.make_async_copy(
                    zeros_hbm.at[pl.ds(0, BATCH)], rows_v.at[j % NBUF],
                    ssems[j % NBUF]).wait()
            return carry

        lax.fori_loop(0, nb // GROUP, group_body, 0)
        plsc.subcore_barrier()
        pltpu.sync_copy(acc_sp.at[pl.ds(r0, rpt)],
                        out_hbm.at[pl.ds(r0, rpt), pl.ds(cid * HALF, HALF)])

    return scat


def _matmul_scale(x_pad, w, deg_col, n_pad):
    bm = 1024

    def body(x_ref, w_ref, deg_ref, o_ref, o16_ref):
        dinv = lax.rsqrt(deg_ref[...])
        h = jnp.dot(x_ref[...], w_ref[...], preferred_element_type=jnp.float32)
        hs = h * dinv
        o_ref[...] = hs
        o16_ref[...] = hs.astype(jnp.bfloat16)

    return pl.pallas_call(
        body,
        grid=(n_pad // bm,),
        in_specs=[
            pl.BlockSpec((bm, 128), lambda i: (i, 0)),
            pl.BlockSpec((128, 128), lambda i: (0, 0)),
            pl.BlockSpec((bm, 1), lambda i: (i, 0)),
        ],
        out_specs=[
            pl.BlockSpec((bm, 128), lambda i: (i, 0)),
            pl.BlockSpec((bm, 128), lambda i: (i, 0)),
        ],
        out_shape=[
            jax.ShapeDtypeStruct((n_pad, 128), jnp.float32),
            jax.ShapeDtypeStruct((n_pad, 128), jnp.bfloat16),
        ],
    )(x_pad, w, deg_col)


def _finalize(acc, hp, deg_col, b_row, n_pad):
    bm = 1024

    def body(a_ref, h_ref, deg_ref, b_ref, o_ref):
        dinv = lax.rsqrt(deg_ref[...])
        s = (a_ref[...].astype(jnp.float32) + h_ref[...]) * dinv + b_ref[...]
        o_ref[...] = jnp.maximum(s, 0.0)

    full_spec = pl.BlockSpec((bm, 128), lambda i: (i, 0))
    return pl.pallas_call(
        body,
        grid=(n_pad // bm,),
        in_specs=[
            full_spec,
            full_spec,
            pl.BlockSpec((bm, 1), lambda i: (i, 0)),
            pl.BlockSpec((1, 128), lambda i: (0, 0)),
        ],
        out_specs=full_spec,
        out_shape=jax.ShapeDtypeStruct((n_pad, 128), jnp.float32),
    )(acc, hp, deg_col, b_row)


def kernel(x, edge_index, W, b):
    n, hidden = x.shape
    e = edge_index.shape[1]
    n_pad = -(-n // (NS * 128)) * (NS * 128)

    src = edge_index[0].astype(jnp.int32)
    dst = edge_index[1].astype(jnp.int32)

    nb = -(-e // (NS * BATCH))
    nb = -(-nb // GROUP) * GROUP
    e_pad = NS * nb * BATCH
    pad_idx = jnp.full((e_pad - e,), n, jnp.int32)
    src2 = jnp.concatenate([src, pad_idx]).reshape(NS * nb, BATCH)
    dst2 = jnp.concatenate([dst, pad_idx]).reshape(NS * nb, BATCH)

    x_pad = jnp.pad(x, ((0, n_pad - n), (0, 0)))
    zeros1 = jnp.zeros((n_pad,), jnp.float32)
    zeros2 = jnp.zeros((n_pad, HALF), jnp.bfloat16)

    hist = _hist_kernel(n_pad, nb // 2)(dst2, zeros1)
    deg_col = (1.0 + hist[:n_pad] + hist[n_pad:]).reshape(n_pad, 1)

    hp, hp16 = _matmul_scale(x_pad, W, deg_col, n_pad)
    acc = _scatter_kernel(n_pad, nb)(hp16, src2, dst2, zeros2)
    out = _finalize(acc, hp, deg_col, b.reshape(1, 128), n_pad)
    return out[:n]


# speedup vs baseline: 3.5032x; 1.0024x over previous
import functools

import jax
import jax.numpy as jnp
from jax import lax
from jax.experimental import pallas as pl
from jax.experimental.pallas import tpu as pltpu
from jax.experimental.pallas import tpu_sc as plsc

NC = 2
NS = 16
HALF = 64
BATCH = 128
GROUP = 160
NBUF = 4
AHEAD = 2


def _mesh():
    return plsc.VectorSubcoreMesh(core_axis_name="c", subcore_axis_name="s")


def _hist_kernel(n_pad, nbh):
    rpt = n_pad // NS

    @functools.partial(
        pl.kernel,
        out_type=jax.ShapeDtypeStruct((NC * n_pad,), jnp.float32),
        mesh=_mesh(),
        scratch_types=[
            pltpu.VMEM((nbh, BATCH), jnp.int32),
            pltpu.VMEM((BATCH,), jnp.float32),
            pltpu.VMEM_SHARED((n_pad,), jnp.float32),
            pltpu.SemaphoreType.DMA,
        ],
    )
    def hist(dst_hbm, zeros_hbm, out_hbm, idx_v, ones_v, deg_sp, hsem):
        cid = lax.axis_index("c")
        sid = lax.axis_index("s")
        wid = sid * NC + cid
        r0 = sid * rpt
        pltpu.sync_copy(zeros_hbm.at[pl.ds(r0, rpt)], deg_sp.at[pl.ds(r0, rpt)])
        for j in range(BATCH // 16):
            ones_v[pl.ds(j * 16, 16)] = jnp.full((16,), 1.0, jnp.float32)
        pltpu.sync_copy(dst_hbm.at[pl.ds(wid * nbh, nbh)], idx_v)
        plsc.subcore_barrier()

        def body(b, carry):
            pltpu.async_copy(ones_v, deg_sp.at[idx_v.at[b]], hsem, add=True)

            @pl.when(b >= 2)
            def _():
                pltpu.make_async_copy(
                    zeros_hbm.at[pl.ds(0, BATCH)], ones_v, hsem).wait()
            return carry

        lax.fori_loop(0, nbh, body, 0)
        for _ in range(2):
            pltpu.make_async_copy(
                zeros_hbm.at[pl.ds(0, BATCH)], ones_v, hsem).wait()
        plsc.subcore_barrier()
        pltpu.sync_copy(deg_sp.at[pl.ds(r0, rpt)],
                        out_hbm.at[pl.ds(cid * n_pad + r0, rpt)])

    return hist


def _scatter_kernel(n_pad, nb):
    rpt = n_pad // NS

    @functools.partial(
        pl.kernel,
        out_type=jax.ShapeDtypeStruct((n_pad, 128), jnp.bfloat16),
        mesh=_mesh(),
        compiler_params=pltpu.CompilerParams(use_tc_tiling_on_sc=False),
        scratch_types=[
            pltpu.VMEM((GROUP, BATCH), jnp.int32),
            pltpu.VMEM((GROUP, BATCH), jnp.int32),
            pltpu.VMEM((NBUF, BATCH, HALF), jnp.bfloat16),
            pltpu.VMEM_SHARED((n_pad, HALF), jnp.bfloat16),
            pltpu.VMEM_SHARED((n_pad, HALF), jnp.bfloat16),
            [pltpu.SemaphoreType.DMA] * NBUF,
            [pltpu.SemaphoreType.DMA] * NBUF,
        ],
    )
    def scat(hp_hbm, src_hbm, dst_hbm, zeros_hbm, out_hbm,
             sidx_v, didx_v, rows_v, h_sp, acc_sp, gsems, ssems):
        cid = lax.axis_index("c")
        sid = lax.axis_index("s")
        r0 = sid * rpt
        pltpu.sync_copy(hp_hbm.at[pl.ds(r0, rpt), pl.ds(cid * HALF, HALF)],
                        h_sp.at[pl.ds(r0, rpt)])
        pltpu.sync_copy(zeros_hbm.at[pl.ds(r0, rpt)], acc_sp.at[pl.ds(r0, rpt)])
        plsc.subcore_barrier()

        def group_body(g, carry):
            base = sid * nb + g * GROUP
            pltpu.sync_copy(src_hbm.at[pl.ds(base, GROUP)], sidx_v)
            pltpu.sync_copy(dst_hbm.at[pl.ds(base, GROUP)], didx_v)
            for k in range(AHEAD):
                pltpu.async_copy(h_sp.at[sidx_v.at[k]], rows_v.at[k], gsems[k])

            def ring(i, c):
                for k in range(NBUF):
                    j = i * NBUF + k
                    pltpu.make_async_copy(
                        h_sp.at[sidx_v.at[j]], rows_v.at[k], gsems[k]).wait()
                    pltpu.async_copy(rows_v.at[k], acc_sp.at[didx_v.at[j]],
                                     ssems[k], add=True)
                    kn = (k + AHEAD) % NBUF

                    @pl.when(j + AHEAD >= NBUF)
                    def _():
                        pltpu.make_async_copy(
                            zeros_hbm.at[pl.ds(0, BATCH)], rows_v.at[kn],
                            ssems[kn]).wait()

                    @pl.when(j + AHEAD < GROUP)
                    def _():
                        pltpu.async_copy(
                            h_sp.at[sidx_v.at[j + AHEAD]], rows_v.at[kn],
                            gsems[kn])
                return c

            lax.fori_loop(0, GROUP // NBUF, ring, 0)
            for j in range(GROUP + AHEAD - NBUF, GROUP):
                pltpu.make_async_copy(
                    zeros_hbm.at[pl.ds(0, BATCH)], rows_v.at[j % NBUF],
                    ssems[j % NBUF]).wait()
            return carry

        lax.fori_loop(0, nb // GROUP, group_body, 0)
        plsc.subcore_barrier()
        pltpu.sync_copy(acc_sp.at[pl.ds(r0, rpt)],
                        out_hbm.at[pl.ds(r0, rpt), pl.ds(cid * HALF, HALF)])

    return scat


def _matmul_scale(x_pad, w, deg_col, n_pad):
    bm = 1024

    def body(x_ref, w_ref, deg_ref, o_ref, o16_ref):
        dinv = lax.rsqrt(deg_ref[...])
        h = jnp.dot(x_ref[...], w_ref[...], preferred_element_type=jnp.float32)
        hs = h * dinv
        o_ref[...] = hs
        o16_ref[...] = hs.astype(jnp.bfloat16)

    return pl.pallas_call(
        body,
        grid=(n_pad // bm,),
        in_specs=[
            pl.BlockSpec((bm, 128), lambda i: (i, 0)),
            pl.BlockSpec((128, 128), lambda i: (0, 0)),
            pl.BlockSpec((bm, 1), lambda i: (i, 0)),
        ],
        out_specs=[
            pl.BlockSpec((bm, 128), lambda i: (i, 0)),
            pl.BlockSpec((bm, 128), lambda i: (i, 0)),
        ],
        out_shape=[
            jax.ShapeDtypeStruct((n_pad, 128), jnp.float32),
            jax.ShapeDtypeStruct((n_pad, 128), jnp.bfloat16),
        ],
    )(x_pad, w, deg_col)


def _finalize(acc, hp, deg_col, b_row, n_pad):
    bm = 1024

    def body(a_ref, h_ref, deg_ref, b_ref, o_ref):
        dinv = lax.rsqrt(deg_ref[...])
        s = (a_ref[...].astype(jnp.float32) + h_ref[...]) * dinv + b_ref[...]
        o_ref[...] = jnp.maximum(s, 0.0)

    full_spec = pl.BlockSpec((bm, 128), lambda i: (i, 0))
    return pl.pallas_call(
        body,
        grid=(n_pad // bm,),
        in_specs=[
            full_spec,
            full_spec,
            pl.BlockSpec((bm, 1), lambda i: (i, 0)),
            pl.BlockSpec((1, 128), lambda i: (0, 0)),
        ],
        out_specs=full_spec,
        out_shape=jax.ShapeDtypeStruct((n_pad, 128), jnp.float32),
    )(acc, hp, deg_col, b_row)


def kernel(x, edge_index, W, b):
    n, hidden = x.shape
    e = edge_index.shape[1]
    n_pad = -(-n // (NS * 128)) * (NS * 128)

    src = edge_index[0].astype(jnp.int32)
    dst = edge_index[1].astype(jnp.int32)

    nb = -(-e // (NS * BATCH))
    nb = -(-nb // GROUP) * GROUP
    e_pad = NS * nb * BATCH
    pad_idx = jnp.full((e_pad - e,), n, jnp.int32)
    src2 = jnp.concatenate([src, pad_idx]).reshape(NS * nb, BATCH)
    dst2 = jnp.concatenate([dst, pad_idx]).reshape(NS * nb, BATCH)

    x_pad = jnp.pad(x, ((0, n_pad - n), (0, 0)))
    zeros1 = jnp.zeros((n_pad,), jnp.float32)
    zeros2 = jnp.zeros((n_pad, HALF), jnp.bfloat16)

    hist = _hist_kernel(n_pad, nb // 2)(dst2, zeros1)
    deg_col = (1.0 + hist[:n_pad] + hist[n_pad:]).reshape(n_pad, 1)

    hp, hp16 = _matmul_scale(x_pad, W, deg_col, n_pad)
    acc = _scatter_kernel(n_pad, nb)(hp16, src2, dst2, zeros2)
    out = _finalize(acc, hp, deg_col, b.reshape(1, 128), n_pad)
    return out[:n]
